# baseline (speedup 1.0000x reference)
import jax
import jax.numpy as jnp
from jax import lax
from jax.experimental import pallas as pl
from jax.experimental.pallas import tpu as pltpu
from jax.experimental.pallas import tpu_sc as plsc

VOCAB = 1000000
D = 64
B = 4096
L = 200
HIDDEN = 128
OUT = 10

NC = 2
NS = 16
NW = NC * NS
BPW = B // NW
NBUF = 3
SPLIT = 128
REST = L - SPLIT
NEG_INF = float("-inf")

BT = 16384
NG = -(-VOCAB // BT)
VOCABP = NG * BT


def _tr_body(in_ref, out_ref):
    t = in_ref[...].T
    out_ref[...] = jnp.concatenate([t, jnp.zeros_like(t)], axis=1)


@jax.jit
def _stage(tableT):
    return pl.pallas_call(
        _tr_body,
        out_shape=jax.ShapeDtypeStruct((VOCABP, 2 * D), jnp.float32),
        grid=(NG,),
        in_specs=[pl.BlockSpec((D, BT), lambda i: (0, i))],
        out_specs=pl.BlockSpec((BT, 2 * D), lambda i: (i, 0)),
    )(tableT)


def _pool_body(x_hbm, table_hbm, out_hbm, idx_v, rows_v, out_v, *sems):
    wid = lax.axis_index("s") * NC + lax.axis_index("c")
    base = wid * BPW

    pltpu.sync_copy(x_hbm.at[pl.ds(base * L, BPW * L)], idx_v)

    def issue(b, p):
        pltpu.async_copy(table_hbm.at[idx_v.at[pl.ds(b * L, SPLIT)]],
                         rows_v.at[p, pl.ds(0, SPLIT), :], sems[p])
        pltpu.async_copy(table_hbm.at[idx_v.at[pl.ds(b * L + SPLIT, REST)]],
                         rows_v.at[p, pl.ds(SPLIT, REST), :], sems[p])

    def wait_buf(p):
        pltpu.make_async_copy(table_hbm.at[pl.ds(0, L), :], rows_v.at[p],
                              sems[p]).wait()

    def accum(carry, p, t, bias):
        s0, s1, s2, s3, m0, m1, m2, m3 = carry
        v0 = rows_v[p, t, pl.ds(0, 16)]
        v1 = rows_v[p, t, pl.ds(16, 16)]
        v2 = rows_v[p, t, pl.ds(32, 16)]
        v3 = rows_v[p, t, pl.ds(48, 16)]
        s0 = s0 + v0
        s1 = s1 + v1
        s2 = s2 + v2
        s3 = s3 + v3
        m0 = jnp.maximum(m0, v0 + bias)
        m1 = jnp.maximum(m1, v1 + bias)
        m2 = jnp.maximum(m2, v2 + bias)
        m3 = jnp.maximum(m3, v3 + bias)
        return s0, s1, s2, s3, m0, m1, m2, m3

    def reduce_row(b, p):
        def g_body(g, carry):
            tv = idx_v[pl.ds(b * L + g * 16, 16)]
            biasv = jnp.where(tv == 0, jnp.float32(NEG_INF), jnp.float32(0.0))
            for j in range(16):
                carry = accum(carry, p, g * 16 + j, biasv[j])
            return carry

        zeros = jnp.zeros((16,), jnp.float32)
        ninf = jnp.full((16,), NEG_INF, jnp.float32)
        init = (zeros, zeros, zeros, zeros, ninf, ninf, ninf, ninf)
        carry = lax.fori_loop(0, L // 16, g_body, init, unroll=1)

        tv = idx_v[pl.ds(b * L + L - 16, 16)]
        biasv = jnp.where(tv == 0, jnp.float32(NEG_INF), jnp.float32(0.0))
        for j in range(16 - (L - L // 16 * 16), 16):
            carry = accum(carry, p, L - 16 + j, biasv[j])

        s0, s1, s2, s3, m0, m1, m2, m3 = carry
        out_v[b, pl.ds(0, 16)] = s0
        out_v[b, pl.ds(16, 16)] = s1
        out_v[b, pl.ds(32, 16)] = s2
        out_v[b, pl.ds(48, 16)] = s3
        out_v[b, pl.ds(64, 16)] = m0
        out_v[b, pl.ds(80, 16)] = m1
        out_v[b, pl.ds(96, 16)] = m2
        out_v[b, pl.ds(112, 16)] = m3

    for p in range(NBUF - 1):
        issue(p, p)

    def outer(i, _):
        for k in range(NBUF):
            r = i * NBUF + k
            wait_buf(k)
            nxt = r + NBUF - 1

            @pl.when(nxt < BPW)
            def _():
                issue(nxt, (k + NBUF - 1) % NBUF)

            reduce_row(r, k)
        return 0

    lax.fori_loop(0, BPW // NBUF, outer, 0)
    rem = BPW - (BPW // NBUF) * NBUF
    for j in range(rem):
        r = (BPW // NBUF) * NBUF + j
        p = r % NBUF
        wait_buf(p)
        reduce_row(r, p)

    pltpu.sync_copy(out_v, out_hbm.at[pl.ds(base, BPW), :])


@jax.jit
def _pool(xf, table):
    mesh = plsc.VectorSubcoreMesh(core_axis_name="c", subcore_axis_name="s",
                                  num_cores=NC, num_subcores=NS)
    return pl.kernel(
        _pool_body,
        out_type=jax.ShapeDtypeStruct((B, 2 * D), jnp.float32),
        mesh=mesh,
        scratch_types=[
            pltpu.VMEM((BPW * L,), jnp.int32),
            pltpu.VMEM((NBUF, L, 2 * D), jnp.float32),
            pltpu.VMEM((BPW, 2 * D), jnp.float32),
        ] + [pltpu.SemaphoreType.DMA] * NBUF,
        compiler_params=pltpu.CompilerParams(use_tc_tiling_on_sc=False),
    )(xf, table)


def _mlp_body(rep_ref, len_ref, w1_ref, b1_ref, w2_ref, b2_ref, out_ref):
    rep = rep_ref[...]
    recip = 1.0 / len_ref[...]
    col = lax.broadcasted_iota(jnp.int32, rep.shape, 1)
    scale = jnp.where(col < D, recip, 1.0)
    h = jnp.dot(rep * scale, w1_ref[...], preferred_element_type=jnp.float32)
    h = jnp.maximum(h + b1_ref[...], 0.0)
    out_ref[...] = (
        jnp.dot(h, w2_ref[...], preferred_element_type=jnp.float32)
        + b2_ref[...])


@jax.jit
def _mlp(reps, lengths_f, w1t, b1, w2t, b2):
    bm = 1024
    return pl.pallas_call(
        _mlp_body,
        out_shape=jax.ShapeDtypeStruct((B, OUT), jnp.float32),
        grid=(B // bm,),
        in_specs=[
            pl.BlockSpec((bm, 2 * D), lambda i: (i, 0)),
            pl.BlockSpec((bm, 1), lambda i: (i, 0)),
            pl.BlockSpec((2 * D, HIDDEN), lambda i: (0, 0)),
            pl.BlockSpec((1, HIDDEN), lambda i: (0, 0)),
            pl.BlockSpec((HIDDEN, OUT), lambda i: (0, 0)),
            pl.BlockSpec((1, OUT), lambda i: (0, 0)),
        ],
        out_specs=pl.BlockSpec((bm, OUT), lambda i: (i, 0)),
    )(reps, lengths_f, w1t, b1, w2t, b2)


def kernel(x, lengths, emb_table, W1, b1, W2, b2):
    staged = _stage(emb_table.T)
    reps = _pool(x.reshape(B * L), staged)
    return _mlp(reps, lengths.astype(jnp.float32).reshape(B, 1), W1.T,
                b1.reshape(1, HIDDEN), W2.T, b2.reshape(1, OUT))


# speedup vs baseline: 1.4868x; 1.4868x over previous
import jax
import jax.numpy as jnp
from jax import lax
from jax.experimental import pallas as pl
from jax.experimental.pallas import tpu as pltpu
from jax.experimental.pallas import tpu_sc as plsc

VOCAB = 1000000
D = 64
B = 4096
L = 200
HIDDEN = 128
OUT = 10

NC = 2
NS = 16
NW = NC * NS
BPW = B // NW
NBUF = 3
SPLIT = 128
REST = L - SPLIT
NEG_INF = float("-inf")

BT = 16384
NG = -(-VOCAB // BT)
VOCABP = NG * BT
HALF = VOCABP // 2


def _tr_body(lo_ref, hi_ref, out_ref):
    out_ref[...] = jnp.concatenate([lo_ref[...].T, hi_ref[...].T], axis=1)


@jax.jit
def _stage(tableT):
    return pl.pallas_call(
        _tr_body,
        out_shape=jax.ShapeDtypeStruct((HALF, 2 * D), jnp.float32),
        grid=(NG // 2,),
        in_specs=[
            pl.BlockSpec((D, BT), lambda i: (0, i)),
            pl.BlockSpec((D, BT), lambda i: (0, i + NG // 2)),
        ],
        out_specs=pl.BlockSpec((BT, 2 * D), lambda i: (i, 0)),
    )(tableT, tableT)


def _pool_body(x_hbm, table_hbm, out_hbm, idx_v, pair_v, rows_v, out_v, *sems):
    wid = lax.axis_index("s") * NC + lax.axis_index("c")
    base = wid * BPW

    pltpu.sync_copy(x_hbm.at[pl.ds(base * L, BPW * L)], idx_v)

    def prep(b, p):
        def mod_body(g, _):
            tv = idx_v[pl.ds(b * L + g * 16, 16)]
            pair_v[pl.ds(p * 256 + g * 16, 16)] = jnp.where(
                tv >= HALF, tv - HALF, tv)
            return 0

        lax.fori_loop(0, L // 16, mod_body, 0, unroll=4)
        tv = idx_v[pl.ds(b * L + L - 16, 16)]
        pair_v[pl.ds(p * 256 + L - 16, 16)] = jnp.where(
            tv >= HALF, tv - HALF, tv)

    def issue(b, p):
        prep(b, p)
        pltpu.async_copy(table_hbm.at[pair_v.at[pl.ds(p * 256, SPLIT)]],
                         rows_v.at[p, pl.ds(0, SPLIT), :], sems[p])
        pltpu.async_copy(table_hbm.at[pair_v.at[pl.ds(p * 256 + SPLIT, REST)]],
                         rows_v.at[p, pl.ds(SPLIT, REST), :], sems[p])

    def wait_buf(p):
        pltpu.make_async_copy(table_hbm.at[pl.ds(0, L), :], rows_v.at[p],
                              sems[p]).wait()

    def accum(carry, p, t, bias, off):
        s0, s1, s2, s3, m0, m1, m2, m3 = carry
        v0 = rows_v[p, t, pl.ds(off, 16)]
        v1 = rows_v[p, t, pl.ds(off + 16, 16)]
        v2 = rows_v[p, t, pl.ds(off + 32, 16)]
        v3 = rows_v[p, t, pl.ds(off + 48, 16)]
        s0 = s0 + v0
        s1 = s1 + v1
        s2 = s2 + v2
        s3 = s3 + v3
        m0 = jnp.maximum(m0, v0 + bias)
        m1 = jnp.maximum(m1, v1 + bias)
        m2 = jnp.maximum(m2, v2 + bias)
        m3 = jnp.maximum(m3, v3 + bias)
        return s0, s1, s2, s3, m0, m1, m2, m3

    def reduce_row(b, p):
        def g_body(g, carry):
            tv = idx_v[pl.ds(b * L + g * 16, 16)]
            biasv = jnp.where(tv == 0, jnp.float32(NEG_INF), jnp.float32(0.0))
            offv = jnp.where(tv >= HALF, jnp.int32(64), jnp.int32(0))
            for j in range(16):
                carry = accum(carry, p, g * 16 + j, biasv[j], offv[j])
            return carry

        zeros = jnp.zeros((16,), jnp.float32)
        ninf = jnp.full((16,), NEG_INF, jnp.float32)
        init = (zeros, zeros, zeros, zeros, ninf, ninf, ninf, ninf)
        carry = lax.fori_loop(0, L // 16, g_body, init, unroll=1)

        tv = idx_v[pl.ds(b * L + L - 16, 16)]
        biasv = jnp.where(tv == 0, jnp.float32(NEG_INF), jnp.float32(0.0))
        offv = jnp.where(tv >= HALF, jnp.int32(64), jnp.int32(0))
        for j in range(16 - (L - L // 16 * 16), 16):
            carry = accum(carry, p, L - 16 + j, biasv[j], offv[j])

        s0, s1, s2, s3, m0, m1, m2, m3 = carry
        out_v[b, pl.ds(0, 16)] = s0
        out_v[b, pl.ds(16, 16)] = s1
        out_v[b, pl.ds(32, 16)] = s2
        out_v[b, pl.ds(48, 16)] = s3
        out_v[b, pl.ds(64, 16)] = m0
        out_v[b, pl.ds(80, 16)] = m1
        out_v[b, pl.ds(96, 16)] = m2
        out_v[b, pl.ds(112, 16)] = m3

    for p in range(NBUF - 1):
        issue(p, p)

    def outer(i, _):
        for k in range(NBUF):
            r = i * NBUF + k
            wait_buf(k)
            nxt = r + NBUF - 1

            @pl.when(nxt < BPW)
            def _():
                issue(nxt, (k + NBUF - 1) % NBUF)

            reduce_row(r, k)
        return 0

    lax.fori_loop(0, BPW // NBUF, outer, 0)
    rem = BPW - (BPW // NBUF) * NBUF
    for j in range(rem):
        r = (BPW // NBUF) * NBUF + j
        p = r % NBUF
        wait_buf(p)
        reduce_row(r, p)

    pltpu.sync_copy(out_v, out_hbm.at[pl.ds(base, BPW), :])


@jax.jit
def _pool(xf, table):
    mesh = plsc.VectorSubcoreMesh(core_axis_name="c", subcore_axis_name="s",
                                  num_cores=NC, num_subcores=NS)
    return pl.kernel(
        _pool_body,
        out_type=jax.ShapeDtypeStruct((B, 2 * D), jnp.float32),
        mesh=mesh,
        scratch_types=[
            pltpu.VMEM((BPW * L,), jnp.int32),
            pltpu.VMEM((NBUF * 256,), jnp.int32),
            pltpu.VMEM((NBUF, L, 2 * D), jnp.float32),
            pltpu.VMEM((BPW, 2 * D), jnp.float32),
        ] + [pltpu.SemaphoreType.DMA] * NBUF,
        compiler_params=pltpu.CompilerParams(use_tc_tiling_on_sc=False),
    )(xf, table)


def _mlp_body(rep_ref, len_ref, w1_ref, b1_ref, w2_ref, b2_ref, out_ref):
    rep = rep_ref[...]
    recip = 1.0 / len_ref[...]
    col = lax.broadcasted_iota(jnp.int32, rep.shape, 1)
    scale = jnp.where(col < D, recip, 1.0)
    h = jnp.dot(rep * scale, w1_ref[...], preferred_element_type=jnp.float32)
    h = jnp.maximum(h + b1_ref[...], 0.0)
    out_ref[...] = (
        jnp.dot(h, w2_ref[...], preferred_element_type=jnp.float32)
        + b2_ref[...])


@jax.jit
def _mlp(reps, lengths_f, w1t, b1, w2t, b2):
    bm = 1024
    return pl.pallas_call(
        _mlp_body,
        out_shape=jax.ShapeDtypeStruct((B, OUT), jnp.float32),
        grid=(B // bm,),
        in_specs=[
            pl.BlockSpec((bm, 2 * D), lambda i: (i, 0)),
            pl.BlockSpec((bm, 1), lambda i: (i, 0)),
            pl.BlockSpec((2 * D, HIDDEN), lambda i: (0, 0)),
            pl.BlockSpec((1, HIDDEN), lambda i: (0, 0)),
            pl.BlockSpec((HIDDEN, OUT), lambda i: (0, 0)),
            pl.BlockSpec((1, OUT), lambda i: (0, 0)),
        ],
        out_specs=pl.BlockSpec((bm, OUT), lambda i: (i, 0)),
    )(reps, lengths_f, w1t, b1, w2t, b2)


def kernel(x, lengths, emb_table, W1, b1, W2, b2):
    staged = _stage(emb_table.T)
    reps = _pool(x.reshape(B * L), staged)
    return _mlp(reps, lengths.astype(jnp.float32).reshape(B, 1), W1.T,
                b1.reshape(1, HIDDEN), W2.T, b2.reshape(1, OUT))
